# baseline (speedup 1.0000x reference)
import functools

import jax
import jax.numpy as jnp
from jax import lax
from jax.experimental import pallas as pl
from jax.experimental.pallas import tpu as pltpu
from jax.experimental.pallas import tpu_sc as plsc

_B = 16384
_F = 26
_V = 100001
_D = 16

_NC = 2
_NS = 16
_FG = _F // 2
_CB = 4096
_NCH = _B // _CB
_H = 49920


def _sc_lookup(idx_t, tab_t):
    mesh = plsc.VectorSubcoreMesh(core_axis_name="c", subcore_axis_name="s")

    @functools.partial(
        pl.kernel,
        mesh=mesh,
        out_type=jax.ShapeDtypeStruct((_F * _D, _B), jnp.float32),
        compiler_params=pltpu.CompilerParams(
            use_tc_tiling_on_sc=True, needs_layout_passes=False
        ),
        scratch_types=[
            pltpu.VMEM((_H, ), jnp.float32),
            pltpu.VMEM((_V - _H,), jnp.float32),
            pltpu.VMEM((_CB,), jnp.int32),
            pltpu.VMEM((_CB,), jnp.int32),
            pltpu.VMEM((_CB,), jnp.float32),
            pltpu.VMEM((_CB,), jnp.float32),
            pltpu.VMEM((_CB,), jnp.float32),
            pltpu.VMEM((_CB,), jnp.float32),
            pltpu.SemaphoreType.DMA,
            pltpu.SemaphoreType.DMA,
            pltpu.SemaphoreType.DMA,
            pltpu.SemaphoreType.DMA,
        ],
    )
    def k(idx_ref, tab_ref, out_ref, rA, rB, idx0, idx1,
          out0, out1, out2, out3, isem, semA, semB, osem):
        wid = lax.axis_index("s") * _NC + lax.axis_index("c")
        d = wid % _D
        f0 = (wid // _D) * _FG
        idxb = (idx0, idx1)
        outb = (out0, out1, out2, out3)

        pltpu.sync_copy(tab_ref.at[f0, d, pl.ds(0, _H)], rA)

        def feat_body(j, carry):
            f = f0 + j
            fnext = f0 + jnp.minimum(j + 1, _FG - 1)
            stB = pltpu.async_copy(tab_ref.at[f, d, pl.ds(_H, _V - _H)],
                                   rB, semB)

            fetches = [
                pltpu.async_copy(idx_ref.at[f, pl.ds(0, _CB)], idx0, isem)
            ]
            for cb in range(_NCH):
                if cb + 1 < _NCH:
                    fetches.append(
                        pltpu.async_copy(
                            idx_ref.at[f, pl.ds((cb + 1) * _CB, _CB)],
                            idxb[(cb + 1) % 2],
                            isem,
                        )
                    )
                fetches.pop(0).wait()
                idx_v = idxb[cb % 2]
                out_v = outb[cb]

                def gather_a(i, carry3):
                    for u in range(8):
                        o = i * 128 + u * 16
                        iv = idx_v[pl.ds(o, 16)]
                        lv = jnp.minimum(iv, _H - 1)
                        out_v[pl.ds(o, 16)] = plsc.load_gather(rA, [lv])
                    return carry3

                lax.fori_loop(0, _CB // 128, gather_a, 0)

            stB.wait()
            stA = pltpu.async_copy(tab_ref.at[fnext, d, pl.ds(0, _H)],
                                   rA, semA)

            fetches = [
                pltpu.async_copy(idx_ref.at[f, pl.ds(0, _CB)], idx0, isem)
            ]
            stores = []
            for cb in range(_NCH):
                if cb + 1 < _NCH:
                    fetches.append(
                        pltpu.async_copy(
                            idx_ref.at[f, pl.ds((cb + 1) * _CB, _CB)],
                            idxb[(cb + 1) % 2],
                            isem,
                        )
                    )
                fetches.pop(0).wait()
                idx_v = idxb[cb % 2]
                out_v = outb[cb]

                def gather_b(i, carry3):
                    for u in range(8):
                        o = i * 128 + u * 16
                        iv = idx_v[pl.ds(o, 16)]
                        lv = iv - _H
                        lvc = jnp.maximum(lv, 0)
                        vals = plsc.load_gather(rB, [lvc])
                        prev = out_v[pl.ds(o, 16)]
                        out_v[pl.ds(o, 16)] = jnp.where(lv >= 0, vals, prev)
                    return carry3

                lax.fori_loop(0, _CB // 128, gather_b, 0)
                stores.append(
                    pltpu.async_copy(
                        out_v,
                        out_ref.at[f * _D + d, pl.ds(cb * _CB, _CB)],
                        osem,
                    )
                )
            for st in stores:
                st.wait()
            stA.wait()
            return carry

        lax.fori_loop(0, _FG, feat_body, 0)

    return k(idx_t, tab_t)


def kernel(indices, tables):
    idx_t = indices.astype(jnp.int32).T
    tab_t = jnp.transpose(tables, (0, 2, 1))
    out_t = _sc_lookup(idx_t, tab_t)
    return out_t.T.reshape(_B, _F * _D)


# speedup vs baseline: 1.3419x; 1.3419x over previous
import functools

import jax
import jax.numpy as jnp
from jax import lax
from jax.experimental import pallas as pl
from jax.experimental.pallas import tpu as pltpu
from jax.experimental.pallas import tpu_sc as plsc

_B = 16384
_F = 26
_V = 100001
_D = 16

_NC = 2
_NS = 16
_FG = _F // 2
_CB = 4096
_NCH = _B // _CB


def _sc_lookup(idx_t, tab_t):
    mesh = plsc.VectorSubcoreMesh(core_axis_name="c", subcore_axis_name="s")

    @functools.partial(
        pl.kernel,
        mesh=mesh,
        out_type=jax.ShapeDtypeStruct((_F * _D, _B), jnp.float32),
        compiler_params=pltpu.CompilerParams(
            use_tc_tiling_on_sc=True, needs_layout_passes=False
        ),
        scratch_types=[
            pltpu.VMEM((_V,), jnp.float32),
            pltpu.VMEM((_CB,), jnp.int32),
            pltpu.VMEM((_CB,), jnp.int32),
            pltpu.VMEM((_CB,), jnp.float32),
            pltpu.VMEM((_CB,), jnp.float32),
            pltpu.SemaphoreType.DMA,
            pltpu.SemaphoreType.DMA,
            pltpu.SemaphoreType.DMA,
        ],
    )
    def k(idx_ref, tab_ref, out_ref, row_v, idx0, idx1, out0, out1,
          isem, osem0, osem1):
        wid = lax.axis_index("s") * _NC + lax.axis_index("c")
        d = wid % _D
        f0 = (wid // _D) * _FG
        idxb = (idx0, idx1)
        outb = (out0, out1)
        osems = (osem0, osem1)

        def feat_body(j, carry):
            f = f0 + j
            pltpu.sync_copy(tab_ref.at[f, d, :], row_v)
            fetches = [
                pltpu.async_copy(idx_ref.at[f, pl.ds(0, _CB)], idx0, isem)
            ]
            stores = []
            for cb in range(_NCH):
                if cb + 1 < _NCH:
                    fetches.append(
                        pltpu.async_copy(
                            idx_ref.at[f, pl.ds((cb + 1) * _CB, _CB)],
                            idxb[(cb + 1) % 2],
                            isem,
                        )
                    )
                fetches.pop(0).wait()
                idx_v = idxb[cb % 2]
                out_v = outb[cb % 2]
                if cb >= 2:
                    stores[cb - 2].wait()

                def gather_body(i, carry3):
                    for u in range(16):
                        o = i * 256 + u * 16
                        iv = idx_v[pl.ds(o, 16)]
                        out_v[pl.ds(o, 16)] = plsc.load_gather(row_v, [iv])
                    return carry3

                lax.fori_loop(0, _CB // 256, gather_body, 0)
                stores.append(
                    pltpu.async_copy(
                        out_v,
                        out_ref.at[f * _D + d, pl.ds(cb * _CB, _CB)],
                        osems[cb % 2],
                    )
                )
            for s in stores[-2:]:
                s.wait()
            return carry

        lax.fori_loop(0, _FG, feat_body, 0)

    return k(idx_t, tab_t)


def kernel(indices, tables):
    idx_t = indices.astype(jnp.int32).T
    tab_t = jnp.transpose(tables, (0, 2, 1))
    out_t = _sc_lookup(idx_t, tab_t)
    return out_t.T.reshape(_B, _F * _D)
